# baseline (speedup 1.0000x reference)
import functools

import jax
import jax.numpy as jnp
from jax import lax
from jax.experimental import pallas as pl
from jax.experimental.pallas import tpu as pltpu
from jax.experimental.pallas import tpu_sc as plsc

_NUM_FIELDS = 26
_VOCAB = 100000
_D_MODEL = 64
_BATCH = 4096

_R = _BATCH * _NUM_FIELDS
_NC, _NS, _L = 2, 16, 16
_NW = _NC * _NS
_RPW = _R // _NW
_CHUNK = 128
_NCHUNK = _RPW // _CHUNK


def _make_sc_gather():
  mesh = plsc.VectorSubcoreMesh(core_axis_name="c", subcore_axis_name="s")

  @functools.partial(
      pl.kernel,
      mesh=mesh,
      out_type=jax.ShapeDtypeStruct((_R, _D_MODEL), jnp.float32),
      scratch_types=[
          pltpu.VMEM((_RPW,), jnp.int32),
          pltpu.VMEM((_CHUNK, _D_MODEL), jnp.float32),
          pltpu.SemaphoreType.DMA,
      ],
      compiler_params=pltpu.CompilerParams(use_tc_tiling_on_sc=False),
  )
  def gather_kernel(tok_hbm, table_hbm, out_hbm, idx_v, rows_v, sem):
    wid = lax.axis_index("s") * _NC + lax.axis_index("c")
    base = wid * _RPW

    pltpu.sync_copy(tok_hbm.at[pl.ds(base, _RPW)], idx_v)

    lanes = lax.iota(jnp.int32, _L)

    def fixup(i, _):
      j = i * _L
      field = (base + j + lanes) % _NUM_FIELDS
      idx_v[pl.ds(j, _L)] = idx_v[pl.ds(j, _L)] + field * _VOCAB
      return 0

    lax.fori_loop(0, _RPW // _L, fixup, 0)

    def chunk(c, _):
      idx_c = idx_v.at[pl.ds(c * _CHUNK, _CHUNK)]
      pltpu.async_copy(table_hbm.at[idx_c], rows_v, sem).wait()
      pltpu.sync_copy(rows_v, out_hbm.at[pl.ds(base + c * _CHUNK, _CHUNK)])
      return 0

    lax.fori_loop(0, _NCHUNK, chunk, 0)

  return gather_kernel


_sc_gather = _make_sc_gather()


@jax.jit
def kernel(context_tokens, tables):
  tok = context_tokens.astype(jnp.int32).reshape(_R)
  table = tables.reshape(_NUM_FIELDS * _VOCAB, _D_MODEL)
  out = _sc_gather(tok, table)
  return out.reshape(_BATCH, _NUM_FIELDS, _D_MODEL)
